# baseline (speedup 1.0000x reference)
import functools

import jax
import jax.numpy as jnp
from jax import lax
from jax.experimental import pallas as pl
from jax.experimental.pallas import tpu as pltpu
from jax.experimental.pallas import tpu_sc as plsc

T_CAP = 128

_NC, _NS = 2, 16
_NW = _NC * _NS


def _pick_block(n: int) -> int:
    for b in (1000, 800, 512, 500, 400, 256, 250, 200, 128, 8):
        if n % b == 0 and b % 8 == 0:
            return b
    return n


def _stats_body(feat_ref, gamma_ref, beta_ref, scale_ref, shift_ref):
    f = feat_ref[...]
    n = f.shape[0]
    mean = jnp.sum(f, axis=0, keepdims=True) * (1.0 / n)
    var = jnp.sum((f - mean) ** 2, axis=0, keepdims=True) * (1.0 / n)
    scale = gamma_ref[...] * jax.lax.rsqrt(var + 1e-5)
    scale_ref[...] = scale
    shift_ref[...] = beta_ref[...] - mean * scale


def _bn_stats(feat, gamma, beta):
    n, d = feat.shape
    return pl.pallas_call(
        _stats_body,
        out_shape=(
            jax.ShapeDtypeStruct((1, d), jnp.float32),
            jax.ShapeDtypeStruct((1, d), jnp.float32),
        ),
    )(feat, gamma.reshape(1, d), beta.reshape(1, d))


def _mailbox_body(epw, k, feat_hbm, ssrc_hbm, offs_hbm, mbox_hbm,
                  idx_v, off_v, rows_v, sem_g, sem_s):
    wid = lax.axis_index("s") * _NC + lax.axis_index("c")
    base = wid * epw

    def step(j, carry):
        b = pl.multiple_of(base + j * k, 8)
        pltpu.sync_copy(ssrc_hbm.at[pl.ds(b, k)], idx_v)
        pltpu.sync_copy(offs_hbm.at[pl.ds(b, k)], off_v)
        pltpu.async_copy(feat_hbm.at[idx_v], rows_v, sem_g).wait()
        pltpu.async_copy(rows_v, mbox_hbm.at[off_v], sem_s).wait()
        return carry

    lax.fori_loop(0, epw // k, step, 0, unroll=False)


def _build_mailbox(feat, ssrc, offs, n, d):
    e = ssrc.shape[0]
    epw = e // _NW
    k = 8
    for cand in (128, 120, 104, 96, 80, 64, 56, 48, 40, 32, 24, 16, 8):
        if epw % cand == 0:
            k = cand
            break
    mesh = plsc.VectorSubcoreMesh(
        core_axis_name="c", subcore_axis_name="s", num_cores=_NC)
    fn = pl.kernel(
        functools.partial(_mailbox_body, epw, k),
        out_type=jax.ShapeDtypeStruct((T_CAP * n + 8, d), jnp.float32),
        mesh=mesh,
        scratch_types=[
            pltpu.VMEM((k,), jnp.int32),
            pltpu.VMEM((k,), jnp.int32),
            pltpu.VMEM((k, d), jnp.float32),
            pltpu.SemaphoreType.DMA,
            pltpu.SemaphoreType.DMA,
        ],
    )
    return fn(feat, ssrc, offs)


def _gru_body(n, blk, h_dim,
              scale_ref, shift_ref, wi_ref, wh_ref, bi_ref, bh_ref,
              ws_ref, wn_ref, deg_ref, feat_ref, mbox_ref, out_ref,
              xbuf, h_ref, sem):
    b = pl.program_id(0)
    deg = deg_ref[...]
    tb = jnp.minimum(jnp.max(deg), T_CAP)
    h_ref[...] = jnp.zeros_like(h_ref)
    row0 = b * blk
    scale = scale_ref[...]
    shift = shift_ref[...]

    def step(t, carry):
        cp = pltpu.make_async_copy(
            mbox_ref.at[pl.ds(pl.multiple_of(t * n + row0, 8), blk), :],
            xbuf, sem)
        cp.start()
        cp.wait()
        x = xbuf[...] * scale + shift
        h = h_ref[...]
        xg = jnp.dot(x, wi_ref[...], preferred_element_type=jnp.float32) + bi_ref[...]
        hg = jnp.dot(h, wh_ref[...], preferred_element_type=jnp.float32) + bh_ref[...]
        r = jax.nn.sigmoid(xg[:, :h_dim] + hg[:, :h_dim])
        z = jax.nn.sigmoid(xg[:, h_dim:2 * h_dim] + hg[:, h_dim:2 * h_dim])
        cand_h = jnp.tanh(xg[:, 2 * h_dim:] + r * hg[:, 2 * h_dim:])
        hnew = (1.0 - z) * cand_h + z * h
        h_ref[...] = jnp.where(t < deg, hnew, h)
        return carry

    lax.fori_loop(0, tb, step, 0, unroll=False)

    fb = feat_ref[...] * scale + shift
    out_ref[...] = (
        jnp.dot(fb, ws_ref[...], preferred_element_type=jnp.float32)
        + jnp.dot(h_ref[...], wn_ref[...], preferred_element_type=jnp.float32))


def _gru_reduce(scale, shift, wi, wh, bi, bh, ws, wn, deg, feat, mbox):
    n, d = feat.shape
    h_dim = wh.shape[0]
    o_dim = ws.shape[1]
    blk = _pick_block(n)
    grid = (n // blk,)
    full = lambda shape: pl.BlockSpec(shape, lambda b: (0,) * len(shape))
    return pl.pallas_call(
        functools.partial(_gru_body, n, blk, h_dim),
        grid=grid,
        in_specs=[
            full((1, d)),
            full((1, d)),
            full((d, 3 * h_dim)),
            full((h_dim, 3 * h_dim)),
            full((1, 3 * h_dim)),
            full((1, 3 * h_dim)),
            full((d, o_dim)),
            full((h_dim, o_dim)),
            pl.BlockSpec((blk, 1), lambda b: (b, 0)),
            pl.BlockSpec((blk, d), lambda b: (b, 0)),
            pl.BlockSpec(memory_space=pl.ANY),
        ],
        out_specs=pl.BlockSpec((blk, o_dim), lambda b: (b, 0)),
        out_shape=jax.ShapeDtypeStruct((n, o_dim), jnp.float32),
        scratch_shapes=[
            pltpu.VMEM((blk, d), jnp.float32),
            pltpu.VMEM((blk, h_dim), jnp.float32),
            pltpu.SemaphoreType.DMA,
        ],
        compiler_params=pltpu.CompilerParams(
            dimension_semantics=("arbitrary",)),
    )(scale, shift, wi, wh, bi, bh, ws, wn, deg, feat, mbox)


def kernel(feat, edge_index, bn_gamma, bn_beta, W_ih, W_hh, b_ih, b_hh,
           W_self, W_neigh):
    n, d = feat.shape
    h_dim = W_hh.shape[1]
    src = edge_index[0]
    dst = edge_index[1]
    e = src.shape[0]

    order = jnp.argsort(dst)
    ssrc = src[order].astype(jnp.int32)
    sdst = dst[order].astype(jnp.int32)
    deg = jnp.bincount(dst, length=n).astype(jnp.int32)
    start = (jnp.cumsum(deg) - deg).astype(jnp.int32)
    t = jnp.arange(e, dtype=jnp.int32) - start[sdst]
    offs = jnp.where(t < T_CAP, t * n + sdst, T_CAP * n).astype(jnp.int32)

    scale, shift = _bn_stats(feat, bn_gamma, bn_beta)
    mbox = _build_mailbox(feat, ssrc, offs, n, d)

    wi = W_ih.T
    wh = W_hh.T
    bi = b_ih.reshape(1, 3 * h_dim)
    bh = b_hh.reshape(1, 3 * h_dim)
    ws = W_self.T
    wn = W_neigh.T
    return _gru_reduce(scale, shift, wi, wh, bi, bh, ws, wn,
                       deg.reshape(n, 1), feat, mbox)


# speedup vs baseline: 1.1611x; 1.1611x over previous
import functools

import jax
import jax.numpy as jnp
from jax import lax
from jax.experimental import pallas as pl
from jax.experimental.pallas import tpu as pltpu
from jax.experimental.pallas import tpu_sc as plsc

T_CAP = 128

CH = 4

_NC, _NS = 2, 16
_NW = _NC * _NS


def _pick_block(n: int) -> int:
    for b in (1000, 800, 512, 500, 400, 256, 250, 200, 128, 8):
        if n % b == 0 and b % 8 == 0:
            return b
    return n


def _stats_body(feat_ref, gamma_ref, beta_ref, scale_ref, shift_ref):
    f = feat_ref[...]
    n = f.shape[0]
    mean = jnp.sum(f, axis=0, keepdims=True) * (1.0 / n)
    var = jnp.sum((f - mean) ** 2, axis=0, keepdims=True) * (1.0 / n)
    scale = gamma_ref[...] * jax.lax.rsqrt(var + 1e-5)
    scale_ref[...] = scale
    shift_ref[...] = beta_ref[...] - mean * scale


def _bn_stats(feat, gamma, beta):
    n, d = feat.shape
    return pl.pallas_call(
        _stats_body,
        out_shape=(
            jax.ShapeDtypeStruct((1, d), jnp.float32),
            jax.ShapeDtypeStruct((1, d), jnp.float32),
        ),
    )(feat, gamma.reshape(1, d), beta.reshape(1, d))


def _mailbox_body(epw, k, feat_hbm, ssrc_hbm, offs_hbm, mbox_hbm,
                  idx_v, off_v, rows_v, sem_g, sem_s):
    wid = lax.axis_index("s") * _NC + lax.axis_index("c")
    base = wid * epw

    def step(j, carry):
        b = pl.multiple_of(base + j * k, 8)
        pltpu.sync_copy(ssrc_hbm.at[pl.ds(b, k)], idx_v)
        pltpu.sync_copy(offs_hbm.at[pl.ds(b, k)], off_v)
        pltpu.async_copy(feat_hbm.at[idx_v], rows_v, sem_g).wait()
        pltpu.async_copy(rows_v, mbox_hbm.at[off_v], sem_s).wait()
        return carry

    lax.fori_loop(0, epw // k, step, 0, unroll=False)


def _build_mailbox(feat, ssrc, offs, n, d):
    e = ssrc.shape[0]
    epw = e // _NW
    k = 8
    for cand in (128, 120, 104, 96, 80, 64, 56, 48, 40, 32, 24, 16, 8):
        if epw % cand == 0:
            k = cand
            break
    mesh = plsc.VectorSubcoreMesh(
        core_axis_name="c", subcore_axis_name="s", num_cores=_NC)
    fn = pl.kernel(
        functools.partial(_mailbox_body, epw, k),
        out_type=jax.ShapeDtypeStruct((T_CAP * n + 8, d), jnp.float32),
        mesh=mesh,
        scratch_types=[
            pltpu.VMEM((k,), jnp.int32),
            pltpu.VMEM((k,), jnp.int32),
            pltpu.VMEM((k, d), jnp.float32),
            pltpu.SemaphoreType.DMA,
            pltpu.SemaphoreType.DMA,
        ],
    )
    return fn(feat, ssrc, offs)


def _gru_body(n, blk, h_dim,
              wi_ref, wh_ref, bix_ref, bh_ref, ws_ref, wn_ref, bout_ref,
              deg_ref, feat_ref, mbox_ref, out_ref,
              xbuf, h_ref, sem):
    b = pl.program_id(0)
    deg = deg_ref[...]
    tb = jnp.minimum(jnp.max(deg), T_CAP)
    tbm1 = jnp.maximum(tb - 1, 0)
    nch = (tb + CH - 1) // CH
    h_ref[...] = jnp.zeros_like(h_ref)
    row0 = b * blk
    bix = bix_ref[...]
    bh = bh_ref[...]

    def start_chunk(c, slot):
        for i in range(CH):
            t = jnp.minimum(c * CH + i, tbm1)
            pltpu.make_async_copy(
                mbox_ref.at[pl.ds(t * n + row0, blk), :],
                xbuf.at[slot, i], sem).start()

    def wait_chunk(slot):
        for i in range(CH):
            pltpu.make_async_copy(
                mbox_ref.at[pl.ds(0, blk), :],
                xbuf.at[slot, i], sem).wait()

    start_chunk(0, 0)

    def chunk_body(c, carry):
        slot = lax.rem(c, 2)
        wait_chunk(slot)
        start_chunk(c + 1, 1 - slot)
        xs = xbuf[slot]
        xb = xs.reshape(CH * blk, xs.shape[-1]).astype(jnp.bfloat16)
        xg = jnp.dot(xb, wi_ref[...],
                     preferred_element_type=jnp.float32)
        xg = xg.reshape(CH, blk, 3 * h_dim) + bix
        for i in range(CH):
            t = c * CH + i
            h = h_ref[...]
            hg = jnp.dot(h.astype(jnp.bfloat16), wh_ref[...],
                         preferred_element_type=jnp.float32) + bh
            xgi = xg[i]
            r = jax.nn.sigmoid(xgi[:, :h_dim] + hg[:, :h_dim])
            z = jax.nn.sigmoid(xgi[:, h_dim:2 * h_dim]
                               + hg[:, h_dim:2 * h_dim])
            cand_h = jnp.tanh(xgi[:, 2 * h_dim:] + r * hg[:, 2 * h_dim:])
            hnew = (1.0 - z) * cand_h + z * h
            h_ref[...] = jnp.where(t < deg, hnew, h)
        return carry

    lax.fori_loop(0, nch, chunk_body, 0, unroll=False)
    wait_chunk(lax.rem(nch, 2))

    fb = feat_ref[...].astype(jnp.bfloat16)
    out_ref[...] = (
        jnp.dot(fb, ws_ref[...], preferred_element_type=jnp.float32)
        + jnp.dot(h_ref[...].astype(jnp.bfloat16), wn_ref[...],
                  preferred_element_type=jnp.float32)
        + bout_ref[...])


def _gru_reduce(wi, wh, bix, bh, ws, wn, bout, deg, feat, mbox):
    n, d = feat.shape
    h_dim = wh.shape[0]
    o_dim = ws.shape[1]
    blk = _pick_block(n)
    grid = (n // blk,)
    full = lambda shape: pl.BlockSpec(shape, lambda b: (0,) * len(shape))
    return pl.pallas_call(
        functools.partial(_gru_body, n, blk, h_dim),
        grid=grid,
        in_specs=[
            full((d, 3 * h_dim)),
            full((h_dim, 3 * h_dim)),
            full((1, 3 * h_dim)),
            full((1, 3 * h_dim)),
            full((d, o_dim)),
            full((h_dim, o_dim)),
            full((1, o_dim)),
            pl.BlockSpec((blk, 1), lambda b: (b, 0)),
            pl.BlockSpec((blk, d), lambda b: (b, 0)),
            pl.BlockSpec(memory_space=pl.ANY),
        ],
        out_specs=pl.BlockSpec((blk, o_dim), lambda b: (b, 0)),
        out_shape=jax.ShapeDtypeStruct((n, o_dim), jnp.float32),
        scratch_shapes=[
            pltpu.VMEM((2, CH, blk, d), jnp.float32),
            pltpu.VMEM((blk, h_dim), jnp.float32),
            pltpu.SemaphoreType.DMA,
        ],
        compiler_params=pltpu.CompilerParams(
            dimension_semantics=("arbitrary",)),
    )(wi, wh, bix, bh, ws, wn, bout, deg, feat, mbox)


def kernel(feat, edge_index, bn_gamma, bn_beta, W_ih, W_hh, b_ih, b_hh,
           W_self, W_neigh):
    n, d = feat.shape
    h_dim = W_hh.shape[1]
    src = edge_index[0]
    dst = edge_index[1]
    e = src.shape[0]

    order = jnp.argsort(dst)
    ssrc = src[order].astype(jnp.int32)
    sdst = dst[order].astype(jnp.int32)
    deg = jnp.bincount(dst, length=n).astype(jnp.int32)
    start = (jnp.cumsum(deg) - deg).astype(jnp.int32)
    t = jnp.arange(e, dtype=jnp.int32) - start[sdst]
    offs = jnp.where(t < T_CAP, t * n + sdst, T_CAP * n).astype(jnp.int32)

    scale, shift = _bn_stats(feat, bn_gamma, bn_beta)
    mbox = _build_mailbox(feat, ssrc, offs, n, d)

    scale_c = scale.reshape(d, 1)
    wi = (scale_c * W_ih.T).astype(jnp.bfloat16)
    wh = W_hh.T.astype(jnp.bfloat16)
    bix = (b_ih.reshape(1, 3 * h_dim)
           + jnp.dot(shift, W_ih.T)).astype(jnp.float32)
    bh = b_hh.reshape(1, 3 * h_dim)
    ws = (scale_c * W_self.T).astype(jnp.bfloat16)
    wn = W_neigh.T.astype(jnp.bfloat16)
    bout = jnp.dot(shift, W_self.T)

    return _gru_reduce(wi, wh, bix, bh, ws, wn, bout,
                       deg.reshape(n, 1), feat, mbox)
